# baseline (speedup 1.0000x reference)
import functools
import math

import jax
import jax.numpy as jnp
from jax import lax
from jax.experimental import pallas as pl
from jax.experimental.pallas import tpu as pltpu
from jax.experimental.pallas import tpu_sc as plsc

VOCAB = 100000
D = 128
B = 4
T = 2048
NC, NS, L = 2, 16, 16
SW = T // NS
PW0, PW1 = 80, 48
NQ0, NQ1 = 5, 2
QW0, QW1 = PW0 // NQ0, PW1 // NQ1
QR0, QR1 = B * QW0, B * QW1
MAXQ = max(NQ0, NQ1)
SCALE = math.sqrt(D)

_mesh = plsc.VectorSubcoreMesh(core_axis_name="c", subcore_axis_name="s")


@functools.partial(
    pl.kernel,
    mesh=_mesh,
    out_type=jax.ShapeDtypeStruct((B, T, D), jnp.float32),
    scratch_types=[
        pltpu.VMEM((MAXQ, 128), jnp.int32),
        pltpu.VMEM((max(NQ0 * QR0, NQ1 * QR1), D), jnp.float32),
        pltpu.VMEM((max(PW0, PW1), D), jnp.float32),
        pltpu.SemaphoreType.DMA,
        pltpu.SemaphoreType.DMA,
        pltpu.SemaphoreType.DMA,
        pltpu.SemaphoreType.DMA,
        pltpu.SemaphoreType.DMA,
        pltpu.SemaphoreType.DMA,
        pltpu.SemaphoreType.DMA,
        pltpu.SemaphoreType.DMA,
    ],
)
def _embed(idx0_hbm, idx1_hbm, tok_hbm, pos_hbm, out_hbm,
           idx_v, rows_v, pos_v, isem, psem, q0, q1, q2, q3, q4, osem):
    s = lax.axis_index("s")
    c = lax.axis_index("c")
    qsems = (q0, q1, q2, q3, q4)

    def run(idx_hbm, pbase, pw, nq, qw, qr):
        pcopy = pltpu.async_copy(
            pos_hbm.at[pl.ds(pbase, pw)], pos_v.at[pl.ds(0, pw)], psem)
        pltpu.async_copy(idx_hbm.at[s], idx_v.at[pl.ds(0, nq)], isem).wait()
        gathers = [
            pltpu.async_copy(
                tok_hbm.at[idx_v.at[q, pl.ds(0, qr)]],
                rows_v.at[pl.ds(q * qr, qr)], qsems[q])
            for q in range(nq)
        ]
        out_waits = []
        for q, g in enumerate(gathers):
            g.wait()
            if q == 0:
                pcopy.wait()

            def body(i, carry, q=q):
                pi = q * qw + i
                for j in range(D // L):
                    sl = pl.ds(j * L, L)
                    pv = pos_v[pi, sl]
                    for b in range(B):
                        row = q * qr + b * qw + i
                        rows_v[row, sl] = rows_v[row, sl] * SCALE + pv
                return carry

            lax.fori_loop(0, qw, body, 0)
            for b in range(B):
                out_waits.append(pltpu.async_copy(
                    rows_v.at[pl.ds(q * qr + b * qw, qw)],
                    out_hbm.at[b, pl.ds(pbase + q * qw, qw)], osem))
        for wt in out_waits:
            wt.wait()

    @pl.when(c == 0)
    def _():
        run(idx0_hbm, s * SW, PW0, NQ0, QW0, QR0)

    @pl.when(c == 1)
    def _():
        run(idx1_hbm, s * SW + PW0, PW1, NQ1, QW1, QR1)


def kernel(token_ids, tok_table, pos_table):
    t = token_ids.astype(jnp.int32).reshape(B, NS, SW)
    idx0 = jnp.pad(
        t[:, :, :PW0].reshape(B, NS, NQ0, QW0).transpose(1, 2, 0, 3)
        .reshape(NS, NQ0, QR0), ((0, 0), (0, 0), (0, 128 - QR0)))
    idx1 = jnp.pad(
        t[:, :, PW0:].reshape(B, NS, NQ1, QW1).transpose(1, 2, 0, 3)
        .reshape(NS, NQ1, QR1), ((0, 0), (0, 0), (0, 128 - QR1)))
    out = _embed(idx0, idx1, tok_table, pos_table)
    return out


# speedup vs baseline: 1.0639x; 1.0639x over previous
import functools
import math

import jax
import jax.numpy as jnp
from jax import lax
from jax.experimental import pallas as pl
from jax.experimental.pallas import tpu as pltpu
from jax.experimental.pallas import tpu_sc as plsc

VOCAB = 100000
D = 128
B = 4
T = 2048
NC, NS, L = 2, 16, 16
NW = NC * NS
PW = T // NW
QWS = (16, 24, 24)
NQ = len(QWS)
QOFF = (0, 16, 40)
QRS = tuple(B * q for q in QWS)
ROFF = tuple(B * o for o in QOFF)
SCALE = math.sqrt(D)

_mesh = plsc.VectorSubcoreMesh(core_axis_name="c", subcore_axis_name="s")


@functools.partial(
    pl.kernel,
    mesh=_mesh,
    out_type=jax.ShapeDtypeStruct((B, T, D), jnp.float32),
    scratch_types=[
        pltpu.VMEM((NQ, 128), jnp.int32),
        pltpu.VMEM((B * PW, D), jnp.float32),
        pltpu.VMEM((PW, D), jnp.float32),
        pltpu.SemaphoreType.DMA,
        pltpu.SemaphoreType.DMA,
    ]
    + [pltpu.SemaphoreType.DMA] * NQ
    + [pltpu.SemaphoreType.DMA],
)
def _embed(idx_hbm, tok_hbm, pos_hbm, out_hbm, idx_v, rows_v, pos_v,
           isem, psem, *rest):
    qsems, osem = rest[:NQ], rest[NQ]
    wid = lax.axis_index("s") * NC + lax.axis_index("c")
    p0 = wid * PW

    pcopy = pltpu.async_copy(pos_hbm.at[pl.ds(p0, PW)], pos_v, psem)
    pltpu.async_copy(idx_hbm.at[wid], idx_v, isem).wait()
    gathers = [
        pltpu.async_copy(
            tok_hbm.at[idx_v.at[q, pl.ds(0, QRS[q])]],
            rows_v.at[pl.ds(ROFF[q], QRS[q])], qsems[q])
        for q in range(NQ)
    ]

    out_waits = []
    for q, g in enumerate(gathers):
        g.wait()
        if q == 0:
            pcopy.wait()

        def body(i, carry, q=q):
            pi = QOFF[q] + i
            for j in range(D // L):
                sl = pl.ds(j * L, L)
                pv = pos_v[pi, sl]
                for b in range(B):
                    row = ROFF[q] + b * QWS[q] + i
                    rows_v[row, sl] = rows_v[row, sl] * SCALE + pv
            return carry

        lax.fori_loop(0, QWS[q], body, 0)
        for b in range(B):
            out_waits.append(pltpu.async_copy(
                rows_v.at[pl.ds(ROFF[q] + b * QWS[q], QWS[q])],
                out_hbm.at[b, pl.ds(p0 + QOFF[q], QWS[q])], osem))

    for wt in out_waits:
        wt.wait()


def kernel(token_ids, tok_table, pos_table):
    t = token_ids.astype(jnp.int32).reshape(B, NW, PW)
    blocks = [
        jnp.pad(t[:, :, QOFF[q]:QOFF[q] + QWS[q]].transpose(1, 0, 2)
                .reshape(NW, QRS[q]), ((0, 0), (0, 128 - QRS[q])))
        for q in range(NQ)
    ]
    idx = jnp.stack(blocks, axis=1)
    out = _embed(idx, tok_table, pos_table)
    return out


# speedup vs baseline: 1.0672x; 1.0031x over previous
import functools
import math

import jax
import jax.numpy as jnp
from jax import lax
from jax.experimental import pallas as pl
from jax.experimental.pallas import tpu as pltpu
from jax.experimental.pallas import tpu_sc as plsc

VOCAB = 100000
D = 128
B = 4
T = 2048
NC, NS, L = 2, 16, 16
NW = NC * NS
PW = T // NW
NQ = 2
QW = PW // NQ
QR = B * QW
SCALE = math.sqrt(D)

_mesh = plsc.VectorSubcoreMesh(core_axis_name="c", subcore_axis_name="s")


@functools.partial(
    pl.kernel,
    mesh=_mesh,
    out_type=jax.ShapeDtypeStruct((B, T, D), jnp.float32),
    scratch_types=[
        pltpu.VMEM((NQ, QR), jnp.int32),
        pltpu.VMEM((NQ * QR, D), jnp.float32),
        pltpu.VMEM((PW, D), jnp.float32),
        pltpu.SemaphoreType.DMA,
        pltpu.SemaphoreType.DMA,
    ]
    + [pltpu.SemaphoreType.DMA] * NQ
    + [pltpu.SemaphoreType.DMA],
)
def _embed(idx_hbm, tok_hbm, pos_hbm, out_hbm, idx_v, rows_v, pos_v,
           isem, psem, *rest):
    qsems, osem = rest[:NQ], rest[NQ]
    wid = lax.axis_index("s") * NC + lax.axis_index("c")
    p0 = wid * PW

    pcopy = pltpu.async_copy(pos_hbm.at[pl.ds(p0, PW)], pos_v, psem)
    pltpu.async_copy(idx_hbm.at[wid], idx_v, isem).wait()
    gathers = [
        pltpu.async_copy(
            tok_hbm.at[idx_v.at[q]],
            rows_v.at[pl.ds(q * QR, QR)], qsems[q])
        for q in range(NQ)
    ]

    out_waits = []
    for q, g in enumerate(gathers):
        g.wait()
        if q == 0:
            pcopy.wait()

        def body(i, carry, q=q):
            pi = q * QW + i
            for j in range(D // L):
                sl = pl.ds(j * L, L)
                pv = pos_v[pi, sl]
                for b in range(B):
                    row = q * QR + b * QW + i
                    rows_v[row, sl] = rows_v[row, sl] * SCALE + pv
            return carry

        lax.fori_loop(0, QW, body, 0)
        for b in range(B):
            out_waits.append(pltpu.async_copy(
                rows_v.at[pl.ds(q * QR + b * QW, QW)],
                out_hbm.at[b, pl.ds(p0 + q * QW, QW)], osem))

    for wt in out_waits:
        wt.wait()


def kernel(token_ids, tok_table, pos_table):
    idx = (token_ids.astype(jnp.int32)
           .reshape(B, NW, NQ, QW)
           .transpose(1, 2, 0, 3)
           .reshape(NW, NQ, QR))
    out = _embed(idx, tok_table, pos_table)
    return out
